# baseline (speedup 1.0000x reference)
import jax
import jax.numpy as jnp
from jax import lax
from jax.experimental import pallas as pl
from jax.experimental.pallas import tpu as pltpu, tpu_sc as plsc

_LANES = 16
_NUM_WORKERS = 16
_NUM_CORES = 1
_UNROLL = 2


def _per_worker(n):
    return -(-n // (_NUM_WORKERS * _LANES)) * _LANES


def _sc_body(energy_hbm, types_hbm, tab_hbm, out_hbm,
             energy_v, types_v, out_v, tab_v, sem0):
    wid = lax.axis_index("s") * _NUM_CORES + lax.axis_index("c")
    n = energy_hbm.shape[0]
    num_types = tab_hbm.shape[0] // 2
    per_w = _per_worker(n)
    base = jnp.minimum(wid * per_w, n - per_w)
    cps = [
        pltpu.make_async_copy(
            types_hbm.at[pl.ds(base, per_w)], types_v, sem0),
        pltpu.make_async_copy(
            energy_hbm.at[pl.ds(base, per_w)], energy_v, sem0),
        pltpu.make_async_copy(tab_hbm, tab_v, sem0),
    ]
    for cp in cps:
        cp.start()
    for cp in cps:
        cp.wait()

    n_vec = per_w // _LANES

    def do_vec(sl):
        t = types_v[sl]
        s = plsc.load_gather(tab_v, [t])
        b = plsc.load_gather(tab_v, [t + num_types])
        out_v[sl] = b + s * energy_v[sl]

    def step(i, carry):
        i0 = i * (_LANES * _UNROLL)
        for u in range(_UNROLL):
            do_vec(pl.ds(i0 + u * _LANES, _LANES))
        return carry

    lax.fori_loop(0, n_vec // _UNROLL, step, 0)
    for v in range(n_vec - n_vec % _UNROLL, n_vec):
        do_vec(pl.ds(v * _LANES, _LANES))

    pltpu.sync_copy(out_v, out_hbm.at[pl.ds(base, per_w)])


def _make_sc_call(n, num_types):
    per_w = _per_worker(n)
    mesh = plsc.VectorSubcoreMesh(
        core_axis_name="c", subcore_axis_name="s", num_cores=_NUM_CORES)
    return pl.kernel(
        _sc_body,
        out_type=jax.ShapeDtypeStruct((n,), jnp.float32),
        mesh=mesh,
        scratch_types=[
            pltpu.VMEM((per_w,), jnp.float32),
            pltpu.VMEM((per_w,), jnp.int32),
            pltpu.VMEM((per_w,), jnp.float32),
            pltpu.VMEM((2 * num_types,), jnp.float32),
            pltpu.SemaphoreType.DMA,
        ],
        compiler_params=pltpu.CompilerParams(
            needs_layout_passes=False,
            disable_bounds_checks=True,
            disable_semaphore_checks=True,
            skip_device_barrier=True,
        ),
    )


def kernel(atomic_energy, atom_types, scales, shifts):
    n = atomic_energy.shape[0]
    x = atomic_energy.reshape(-1)
    t = atom_types.reshape(-1).astype(jnp.int32)
    tab = jnp.concatenate([scales, shifts])
    num_types = scales.shape[0]
    per_w = _per_worker(n)
    if n < per_w or n % _LANES or (n - per_w) % 8:
        pad = _NUM_WORKERS * per_w - n
        x = jnp.pad(x, (0, pad))
        t = jnp.pad(t, (0, pad))
        out = _make_sc_call(x.shape[0], num_types)(x, t, tab)
        return out[:n].reshape(-1, 1)
    out = _make_sc_call(n, num_types)(x, t, tab)
    return out.reshape(-1, 1)


# speedup vs baseline: 1.0520x; 1.0520x over previous
import jax
import jax.numpy as jnp
from jax import lax
from jax.experimental import pallas as pl
from jax.experimental.pallas import tpu as pltpu, tpu_sc as plsc

_LANES = 16
_NUM_WORKERS = 16
_NUM_CORES = 1
_UNROLL = 2


def _per_worker(n):
    return -(-n // (_NUM_WORKERS * _LANES)) * _LANES


def _sc_body(energy_hbm, types_hbm, scales_hbm, shifts_hbm, out_hbm,
             energy_v, types_v, out_v, scales_v, shifts_v, sem0):
    wid = lax.axis_index("s") * _NUM_CORES + lax.axis_index("c")
    n = energy_hbm.shape[0]
    per_w = _per_worker(n)
    base = jnp.minimum(wid * per_w, n - per_w)
    cps = [
        pltpu.make_async_copy(
            types_hbm.at[pl.ds(base, per_w)], types_v, sem0),
        pltpu.make_async_copy(
            energy_hbm.at[pl.ds(base, per_w)], energy_v, sem0),
        pltpu.make_async_copy(scales_hbm, scales_v, sem0),
        pltpu.make_async_copy(shifts_hbm, shifts_v, sem0),
    ]
    for cp in cps:
        cp.start()
    for cp in cps:
        cp.wait()

    n_vec = per_w // _LANES

    def do_vec(sl):
        t = types_v[sl]
        s = plsc.load_gather(scales_v, [t])
        b = plsc.load_gather(shifts_v, [t])
        out_v[sl] = b + s * energy_v[sl]

    def step(i, carry):
        i0 = i * (_LANES * _UNROLL)
        for u in range(_UNROLL):
            do_vec(pl.ds(i0 + u * _LANES, _LANES))
        return carry

    lax.fori_loop(0, n_vec // _UNROLL, step, 0)
    for v in range(n_vec - n_vec % _UNROLL, n_vec):
        do_vec(pl.ds(v * _LANES, _LANES))

    pltpu.sync_copy(out_v, out_hbm.at[pl.ds(base, per_w)])


def _make_sc_call(n, num_types):
    per_w = _per_worker(n)
    mesh = plsc.VectorSubcoreMesh(
        core_axis_name="c", subcore_axis_name="s", num_cores=_NUM_CORES)
    return pl.kernel(
        _sc_body,
        out_type=jax.ShapeDtypeStruct((n,), jnp.float32),
        mesh=mesh,
        scratch_types=[
            pltpu.VMEM((per_w,), jnp.float32),
            pltpu.VMEM((per_w,), jnp.int32),
            pltpu.VMEM((per_w,), jnp.float32),
            pltpu.VMEM((num_types,), jnp.float32),
            pltpu.VMEM((num_types,), jnp.float32),
            pltpu.SemaphoreType.DMA,
        ],
        compiler_params=pltpu.CompilerParams(
            needs_layout_passes=False,
            disable_bounds_checks=True,
            disable_semaphore_checks=True,
            skip_device_barrier=True,
        ),
    )


def kernel(atomic_energy, atom_types, scales, shifts):
    n = atomic_energy.shape[0]
    x = atomic_energy.reshape(-1)
    t = atom_types.reshape(-1).astype(jnp.int32)
    num_types = scales.shape[0]
    per_w = _per_worker(n)
    if n < per_w or n % _LANES or (n - per_w) % 8:
        pad = _NUM_WORKERS * per_w - n
        x = jnp.pad(x, (0, pad))
        t = jnp.pad(t, (0, pad))
        out = _make_sc_call(x.shape[0], num_types)(x, t, scales, shifts)
        return out[:n].reshape(-1, 1)
    out = _make_sc_call(n, num_types)(x, t, scales, shifts)
    return out.reshape(-1, 1)


# speedup vs baseline: 1.0527x; 1.0006x over previous
import jax
import jax.numpy as jnp
from jax import lax
from jax.experimental import pallas as pl
from jax.experimental.pallas import tpu as pltpu, tpu_sc as plsc

_LANES = 16
_NUM_WORKERS = 16
_NUM_CORES = 1
_UNROLL = 2


def _per_worker(n):
    return -(-n // (_NUM_WORKERS * _LANES)) * _LANES


def _sc_body(energy_hbm, types_hbm, scales_hbm, shifts_hbm, out_hbm,
             energy_v, types_v, out_v, scales_v, shifts_v, sem0):
    wid = lax.axis_index("s") * _NUM_CORES + lax.axis_index("c")
    n = energy_hbm.shape[0]
    per_w = _per_worker(n)
    base = jnp.minimum(wid * per_w, n - per_w)
    cps = [
        pltpu.make_async_copy(
            types_hbm.at[pl.ds(base, per_w)], types_v, sem0),
        pltpu.make_async_copy(
            energy_hbm.at[pl.ds(base, per_w)], energy_v, sem0),
        pltpu.make_async_copy(scales_hbm, scales_v, sem0),
        pltpu.make_async_copy(shifts_hbm, shifts_v, sem0),
    ]
    for cp in cps:
        cp.start()
    for cp in cps:
        cp.wait()

    n_vec = per_w // _LANES

    def do_vec(sl):
        t = types_v[sl]
        s = plsc.load_gather(scales_v, [t])
        b = plsc.load_gather(shifts_v, [t])
        out_v[sl] = b + s * energy_v[sl]

    def step(i, carry):
        i0 = i * (_LANES * _UNROLL)
        for u in range(_UNROLL):
            do_vec(pl.ds(i0 + u * _LANES, _LANES))
        return carry

    lax.fori_loop(0, n_vec // _UNROLL, step, 0)
    for v in range(n_vec - n_vec % _UNROLL, n_vec):
        do_vec(pl.ds(v * _LANES, _LANES))

    pltpu.sync_copy(out_v, out_hbm.at[pl.ds(base, per_w)])


def _make_sc_call(n, num_types):
    per_w = _per_worker(n)
    mesh = plsc.VectorSubcoreMesh(
        core_axis_name="c", subcore_axis_name="s", num_cores=_NUM_CORES)
    return pl.kernel(
        _sc_body,
        out_type=jax.ShapeDtypeStruct((n,), jnp.float32),
        mesh=mesh,
        scratch_types=[
            pltpu.VMEM((per_w,), jnp.float32),
            pltpu.VMEM((per_w,), jnp.int32),
            pltpu.VMEM((per_w,), jnp.float32),
            pltpu.VMEM((num_types,), jnp.float32),
            pltpu.VMEM((num_types,), jnp.float32),
            pltpu.SemaphoreType.DMA,
        ],
        compiler_params=pltpu.CompilerParams(needs_layout_passes=False),
    )


def kernel(atomic_energy, atom_types, scales, shifts):
    n = atomic_energy.shape[0]
    x = atomic_energy.reshape(-1)
    t = atom_types.reshape(-1).astype(jnp.int32)
    num_types = scales.shape[0]
    per_w = _per_worker(n)
    if n < per_w or n % _LANES or (n - per_w) % 8:
        pad = _NUM_WORKERS * per_w - n
        x = jnp.pad(x, (0, pad))
        t = jnp.pad(t, (0, pad))
        out = _make_sc_call(x.shape[0], num_types)(x, t, scales, shifts)
        return out[:n].reshape(-1, 1)
    out = _make_sc_call(n, num_types)(x, t, scales, shifts)
    return out.reshape(-1, 1)


# speedup vs baseline: 1.0531x; 1.0004x over previous
import jax
import jax.numpy as jnp
from jax import lax
from jax.experimental import pallas as pl
from jax.experimental.pallas import tpu as pltpu, tpu_sc as plsc

_LANES = 16
_NUM_WORKERS = 16
_NUM_CORES = 1
_UNROLL = 2


def _per_worker(n):
    return -(-n // (_NUM_WORKERS * _LANES)) * _LANES


def _sc_body(energy_hbm, types_hbm, scales_hbm, shifts_hbm, out_hbm,
             energy_v, types_v, out_v, scales_v, shifts_v, sem0, sem1):
    wid = lax.axis_index("s") * _NUM_CORES + lax.axis_index("c")
    n = energy_hbm.shape[0]
    per_w = _per_worker(n)
    base = jnp.minimum(wid * per_w, n - per_w)
    cps = [
        pltpu.make_async_copy(
            types_hbm.at[pl.ds(base, per_w)], types_v, sem0),
        pltpu.make_async_copy(
            energy_hbm.at[pl.ds(base, per_w)], energy_v, sem0),
        pltpu.make_async_copy(scales_hbm, scales_v, sem0),
        pltpu.make_async_copy(shifts_hbm, shifts_v, sem0),
    ]
    for cp in cps:
        cp.start()
    for cp in cps:
        cp.wait()

    n_vec = per_w // _LANES

    def do_vec(sl):
        t = types_v[sl]
        s = plsc.load_gather(scales_v, [t])
        b = plsc.load_gather(shifts_v, [t])
        out_v[sl] = b + s * energy_v[sl]

    def step(i, carry):
        i0 = i * (_LANES * _UNROLL)
        for u in range(_UNROLL):
            do_vec(pl.ds(i0 + u * _LANES, _LANES))
        return carry

    half_vec = n_vec // 2 // _UNROLL * _UNROLL
    half = half_vec * _LANES
    lax.fori_loop(0, half_vec // _UNROLL, step, 0)
    out0 = pltpu.make_async_copy(
        out_v.at[pl.ds(0, half)], out_hbm.at[pl.ds(base, half)], sem1)
    out0.start()
    lax.fori_loop(half_vec // _UNROLL, n_vec // _UNROLL, step, 0)
    for v in range(n_vec - n_vec % _UNROLL, n_vec):
        do_vec(pl.ds(v * _LANES, _LANES))
    pltpu.sync_copy(
        out_v.at[pl.ds(half, per_w - half)],
        out_hbm.at[pl.ds(base + half, per_w - half)])
    out0.wait()


def _make_sc_call(n, num_types):
    per_w = _per_worker(n)
    mesh = plsc.VectorSubcoreMesh(
        core_axis_name="c", subcore_axis_name="s", num_cores=_NUM_CORES)
    return pl.kernel(
        _sc_body,
        out_type=jax.ShapeDtypeStruct((n,), jnp.float32),
        mesh=mesh,
        scratch_types=[
            pltpu.VMEM((per_w,), jnp.float32),
            pltpu.VMEM((per_w,), jnp.int32),
            pltpu.VMEM((per_w,), jnp.float32),
            pltpu.VMEM((num_types,), jnp.float32),
            pltpu.VMEM((num_types,), jnp.float32),
            pltpu.SemaphoreType.DMA,
            pltpu.SemaphoreType.DMA,
        ],
        compiler_params=pltpu.CompilerParams(needs_layout_passes=False),
    )


def kernel(atomic_energy, atom_types, scales, shifts):
    n = atomic_energy.shape[0]
    x = atomic_energy.reshape(-1)
    t = atom_types.reshape(-1).astype(jnp.int32)
    num_types = scales.shape[0]
    per_w = _per_worker(n)
    if n < per_w or n % _LANES or (n - per_w) % 8:
        pad = _NUM_WORKERS * per_w - n
        x = jnp.pad(x, (0, pad))
        t = jnp.pad(t, (0, pad))
        out = _make_sc_call(x.shape[0], num_types)(x, t, scales, shifts)
        return out[:n].reshape(-1, 1)
    out = _make_sc_call(n, num_types)(x, t, scales, shifts)
    return out.reshape(-1, 1)


# speedup vs baseline: 1.0550x; 1.0018x over previous
import jax
import jax.numpy as jnp
from jax import lax
from jax.experimental import pallas as pl
from jax.experimental.pallas import tpu as pltpu, tpu_sc as plsc

_LANES = 16
_NUM_WORKERS = 16
_NUM_CORES = 1
_UNROLL = 2


def _per_worker(n):
    return -(-n // (_NUM_WORKERS * _LANES)) * _LANES


def _sc_body(energy_hbm, types_hbm, scales_hbm, shifts_hbm, out_hbm,
             energy_v, types_v, out_v, scales_v, shifts_v, sem0):
    wid = lax.axis_index("s") * _NUM_CORES + lax.axis_index("c")
    n = energy_hbm.shape[0]
    per_w = _per_worker(n)
    base = jnp.minimum(wid * per_w, n - per_w)
    cps = [
        pltpu.make_async_copy(
            types_hbm.at[pl.ds(base, per_w)], types_v, sem0),
        pltpu.make_async_copy(
            energy_hbm.at[pl.ds(base, per_w)], energy_v, sem0),
        pltpu.make_async_copy(scales_hbm, scales_v, sem0),
        pltpu.make_async_copy(shifts_hbm, shifts_v, sem0),
    ]
    for cp in cps:
        cp.start()
    for cp in cps:
        cp.wait()

    n_vec = per_w // _LANES

    def do_vec(sl):
        t = types_v[sl]
        s = plsc.load_gather(scales_v, [t])
        b = plsc.load_gather(shifts_v, [t])
        out_v[sl] = b + s * energy_v[sl]

    def step(i, carry):
        i0 = i * (_LANES * _UNROLL)
        for u in range(_UNROLL):
            do_vec(pl.ds(i0 + u * _LANES, _LANES))
        return carry

    lax.fori_loop(0, n_vec // _UNROLL, step, 0)
    for v in range(n_vec - n_vec % _UNROLL, n_vec):
        do_vec(pl.ds(v * _LANES, _LANES))

    pltpu.sync_copy(out_v, out_hbm.at[pl.ds(base, per_w)])


def _make_sc_call(n, num_types):
    per_w = _per_worker(n)
    mesh = plsc.VectorSubcoreMesh(
        core_axis_name="c", subcore_axis_name="s", num_cores=_NUM_CORES)
    return pl.kernel(
        _sc_body,
        out_type=jax.ShapeDtypeStruct((n,), jnp.float32),
        mesh=mesh,
        scratch_types=[
            pltpu.VMEM((per_w,), jnp.float32),
            pltpu.VMEM((per_w,), jnp.int32),
            pltpu.VMEM((per_w,), jnp.float32),
            pltpu.VMEM((num_types,), jnp.float32),
            pltpu.VMEM((num_types,), jnp.float32),
            pltpu.SemaphoreType.DMA,
        ],
        compiler_params=pltpu.CompilerParams(needs_layout_passes=False),
    )


def kernel(atomic_energy, atom_types, scales, shifts):
    n = atomic_energy.shape[0]
    x = atomic_energy.reshape(-1)
    t = atom_types.reshape(-1).astype(jnp.int32)
    num_types = scales.shape[0]
    per_w = _per_worker(n)
    if n < per_w or n % _LANES or (n - per_w) % 8:
        pad = _NUM_WORKERS * per_w - n
        x = jnp.pad(x, (0, pad))
        t = jnp.pad(t, (0, pad))
        out = _make_sc_call(x.shape[0], num_types)(x, t, scales, shifts)
        return out[:n].reshape(-1, 1)
    out = _make_sc_call(n, num_types)(x, t, scales, shifts)
    return out.reshape(-1, 1)
